# Initial kernel scaffold; baseline (speedup 1.0000x reference)
import jax
import jax.numpy as jnp
from jax.experimental import pallas as pl


def kernel(feat, edge_index, weight, bias):
    raise NotImplementedError("write your pallas kernel here")


# speedup vs baseline: 6.4095x; 6.4095x over previous
import functools

import jax
import jax.numpy as jnp
from jax import lax
from jax.experimental import pallas as pl
from jax.experimental.pallas import tpu as pltpu
from jax.experimental.pallas import tpu_sc as plsc

N = 10000
NPAD = 10240
E = 320000
D = 128
NC = 2
NS = 16
NW = NC * NS
EW = E // NW
K = 80
NCH = EW // K
SL = NPAD // NS


def _mesh():
    return plsc.VectorSubcoreMesh(
        core_axis_name="c", subcore_axis_name="s", num_cores=NC, num_subcores=NS
    )


def _deg_body(src3, dst3, zeros1, degp, sidx, didx, ones, sh_out, sh_in):
    c = lax.axis_index("c")
    s = lax.axis_index("s")
    wid = s * NC + c
    pltpu.sync_copy(zeros1, sh_out.at[pl.ds(s * SL, SL)])
    pltpu.sync_copy(zeros1, sh_in.at[pl.ds(s * SL, SL)])
    for i in range(K // 16):
        ones[pl.ds(i * 16, 16)] = jnp.ones((16,), jnp.float32)
    pltpu.sync_copy(src3.at[wid], sidx)
    pltpu.sync_copy(dst3.at[wid], didx)
    plsc.subcore_barrier()

    def chunk(j, carry):
        pltpu.sync_copy(ones, sh_out.at[sidx.at[j]], add=True)
        pltpu.sync_copy(ones, sh_in.at[didx.at[j]], add=True)
        return carry

    lax.fori_loop(0, NCH, chunk, 0)
    plsc.subcore_barrier()
    pltpu.sync_copy(sh_out.at[pl.ds(s * SL, SL)], degp.at[0, c, pl.ds(s * SL, SL)])
    pltpu.sync_copy(sh_in.at[pl.ds(s * SL, SL)], degp.at[1, c, pl.ds(s * SL, SL)])


def _spmm_body(table, src3, dst3, zrows, aggp, sidx, didx, rows, sh_acc, sem):
    c = lax.axis_index("c")
    s = lax.axis_index("s")
    wid = s * NC + c
    pltpu.sync_copy(zrows, sh_acc.at[pl.ds(s * SL, SL)])
    pltpu.sync_copy(src3.at[wid], sidx)
    pltpu.sync_copy(dst3.at[wid], didx)
    plsc.subcore_barrier()

    def chunk(j, carry):
        pltpu.async_copy(table.at[sidx.at[j]], rows, sem).wait()
        pltpu.sync_copy(rows, sh_acc.at[didx.at[j]], add=True)
        return carry

    lax.fori_loop(0, NCH, chunk, 0)
    plsc.subcore_barrier()
    pltpu.sync_copy(sh_acc.at[pl.ds(s * SL, SL)], aggp.at[c, pl.ds(s * SL, SL)])


def _src_norm_body(f_ref, d_ref, o_ref):
    deg = d_ref[0, pl.ds(0, N), :] + d_ref[1, pl.ds(0, N), :]
    norm = lax.rsqrt(jnp.maximum(deg, 1.0))
    o_ref[...] = f_ref[...] * norm


def _out_body(a_ref, w_ref, d_ref, b_ref, o_ref):
    agg = a_ref[0, pl.ds(0, N), :] + a_ref[1, pl.ds(0, N), :]
    rst = jnp.dot(agg, w_ref[...], preferred_element_type=jnp.float32)
    deg = d_ref[0, pl.ds(0, N), :] + d_ref[1, pl.ds(0, N), :]
    norm = lax.rsqrt(jnp.maximum(deg, 1.0))
    o_ref[...] = rst * norm + b_ref[...]


@jax.jit
def kernel(feat, edge_index, weight, bias):
    src3 = edge_index[0].astype(jnp.int32).reshape(NW, NCH, K)
    dst3 = edge_index[1].astype(jnp.int32).reshape(NW, NCH, K)
    zeros1 = jnp.zeros((SL,), jnp.float32)
    zrows = jnp.zeros((SL, D), jnp.float32)

    degp = pl.kernel(
        _deg_body,
        out_type=jax.ShapeDtypeStruct((2, NC, NPAD), jnp.float32),
        mesh=_mesh(),
        scratch_types=[
            pltpu.VMEM((NCH, K), jnp.int32),
            pltpu.VMEM((NCH, K), jnp.int32),
            pltpu.VMEM((K,), jnp.float32),
            pltpu.VMEM_SHARED((NPAD,), jnp.float32),
            pltpu.VMEM_SHARED((NPAD,), jnp.float32),
        ],
    )(src3, dst3, zeros1)

    feat_src = pl.pallas_call(
        _src_norm_body,
        out_shape=jax.ShapeDtypeStruct((N, D), jnp.float32),
    )(feat, degp[0][..., None])

    aggp = pl.kernel(
        _spmm_body,
        out_type=jax.ShapeDtypeStruct((NC, NPAD, D), jnp.float32),
        mesh=_mesh(),
        scratch_types=[
            pltpu.VMEM((NCH, K), jnp.int32),
            pltpu.VMEM((NCH, K), jnp.int32),
            pltpu.VMEM((K, D), jnp.float32),
            pltpu.VMEM_SHARED((NPAD, D), jnp.float32),
            pltpu.SemaphoreType.DMA,
        ],
    )(feat_src, src3, dst3, zrows)

    out = pl.pallas_call(
        _out_body,
        out_shape=jax.ShapeDtypeStruct((N, D), jnp.float32),
    )(aggp, weight, degp[1][..., None], bias.reshape(1, D))
    return out


# speedup vs baseline: 10.5942x; 1.6529x over previous
import jax
import jax.numpy as jnp
from jax import lax
from jax.experimental import pallas as pl
from jax.experimental.pallas import tpu as pltpu
from jax.experimental.pallas import tpu_sc as plsc

N = 10000
NPAD = 10240
E = 320000
D = 128
NC = 2
NS = 16
NW = NC * NS
EW = E // NW
K = 80
NCH = EW // K
SL = NPAD // NS
NBUF = 3
NIB = 2 * NBUF
NMAIN = (NCH // NIB) * NIB


def _mesh():
    return plsc.VectorSubcoreMesh(
        core_axis_name="c", subcore_axis_name="s", num_cores=NC, num_subcores=NS
    )


def _fill_ones(ones):
    for i in range(K // 16):
        ones[pl.ds(i * 16, 16)] = jnp.ones((16,), jnp.float32)


def _deg_body(src3, dst3, zeros1, degp, sidx, didx, ones, sh_out, sh_in, sem):
    c = lax.axis_index("c")
    s = lax.axis_index("s")
    wid = s * NC + c
    pltpu.sync_copy(zeros1, sh_out.at[pl.ds(s * SL, SL)])
    pltpu.sync_copy(zeros1, sh_in.at[pl.ds(s * SL, SL)])
    _fill_ones(ones)
    pltpu.sync_copy(src3.at[wid], sidx)
    pltpu.sync_copy(dst3.at[wid], didx)
    plsc.subcore_barrier()

    def chunk(j, carry):
        pltpu.async_copy(ones, sh_out.at[sidx.at[j]], sem, add=True)
        pltpu.async_copy(ones, sh_in.at[didx.at[j]], sem, add=True)
        pltpu.make_async_copy(ones, sh_out.at[sidx.at[j]], sem).wait()
        pltpu.make_async_copy(ones, sh_in.at[didx.at[j]], sem).wait()
        return carry

    lax.fori_loop(0, NCH, chunk, 0)
    plsc.subcore_barrier()
    pltpu.sync_copy(sh_out.at[pl.ds(s * SL, SL)], degp.at[0, c, pl.ds(s * SL, SL)])
    pltpu.sync_copy(sh_in.at[pl.ds(s * SL, SL)], degp.at[1, c, pl.ds(s * SL, SL)])


def _spmm_body(table, src3, dst3, zrows, aggp, sbuf, dbuf, rows, sh_acc,
               isem, gsem):
    c = lax.axis_index("c")
    s = lax.axis_index("s")
    wid = s * NC + c
    pltpu.sync_copy(zrows, sh_acc.at[pl.ds(s * SL, SL)])
    plsc.subcore_barrier()

    def fire_idx(j, slot):
        pltpu.async_copy(src3.at[wid, j], sbuf.at[slot], isem.at[slot])
        pltpu.async_copy(dst3.at[wid, j], dbuf.at[slot], isem.at[slot])

    def wait_idx(slot):
        pltpu.make_async_copy(src3.at[wid, 0], sbuf.at[slot], isem.at[slot]).wait()
        pltpu.make_async_copy(src3.at[wid, 0], dbuf.at[slot], isem.at[slot]).wait()

    def fire_gather(slot, b):
        pltpu.async_copy(table.at[sbuf.at[slot]], rows.at[b], gsem.at[b])

    def wait_gather(b):
        pltpu.make_async_copy(table.at[pl.ds(0, K)], rows.at[b], gsem.at[b]).wait()

    for ib in range(NIB):
        fire_idx(ib, ib)
    for b in range(NBUF):
        wait_idx(b)
        fire_gather(b, b)

    def outer(g, carry):
        for ib in range(NIB):
            b = ib % NBUF
            j = g * NIB + ib
            wait_gather(b)
            pltpu.sync_copy(rows.at[b], sh_acc.at[dbuf.at[ib]], add=True)

            @pl.when(j + NIB < NCH)
            def _refill():
                fire_idx(j + NIB, ib)

            nslot = (ib + NBUF) % NIB
            wait_idx(nslot)
            fire_gather(nslot, b)
        return carry

    lax.fori_loop(0, NMAIN // NIB, outer, 0)

    for t in range(NCH - NMAIN):
        j = NMAIN + t
        ib = j % NIB
        b = ib % NBUF
        wait_gather(b)
        pltpu.sync_copy(rows.at[b], sh_acc.at[dbuf.at[ib]], add=True)
        nxt = j + NBUF
        if nxt < NCH:
            nslot = nxt % NIB
            wait_idx(nslot)
            fire_gather(nslot, b)

    plsc.subcore_barrier()
    pltpu.sync_copy(sh_acc.at[pl.ds(s * SL, SL)], aggp.at[c, pl.ds(s * SL, SL)])


def _src_norm_body(f_ref, d_ref, o_ref):
    deg = d_ref[0, pl.ds(0, N), :] + d_ref[1, pl.ds(0, N), :]
    norm = lax.rsqrt(jnp.maximum(deg, 1.0))
    o_ref[...] = f_ref[...] * norm


def _out_body(a_ref, w_ref, d_ref, b_ref, o_ref):
    agg = a_ref[0, pl.ds(0, N), :] + a_ref[1, pl.ds(0, N), :]
    rst = jnp.dot(agg, w_ref[...], preferred_element_type=jnp.float32)
    deg = d_ref[0, pl.ds(0, N), :] + d_ref[1, pl.ds(0, N), :]
    norm = lax.rsqrt(jnp.maximum(deg, 1.0))
    o_ref[...] = rst * norm + b_ref[...]


@jax.jit
def kernel(feat, edge_index, weight, bias):
    src3 = edge_index[0].astype(jnp.int32).reshape(NW, NCH, K)
    dst3 = edge_index[1].astype(jnp.int32).reshape(NW, NCH, K)
    zeros1 = jnp.zeros((SL,), jnp.float32)
    zrows = jnp.zeros((SL, D), jnp.float32)

    degp = pl.kernel(
        _deg_body,
        out_type=jax.ShapeDtypeStruct((2, NC, NPAD), jnp.float32),
        mesh=_mesh(),
        scratch_types=[
            pltpu.VMEM((NCH, K), jnp.int32),
            pltpu.VMEM((NCH, K), jnp.int32),
            pltpu.VMEM((K,), jnp.float32),
            pltpu.VMEM_SHARED((NPAD,), jnp.float32),
            pltpu.VMEM_SHARED((NPAD,), jnp.float32),
            pltpu.SemaphoreType.DMA,
        ],
    )(src3, dst3, zeros1)

    feat_src = pl.pallas_call(
        _src_norm_body,
        out_shape=jax.ShapeDtypeStruct((N, D), jnp.float32),
    )(feat, degp[0][..., None])

    aggp = pl.kernel(
        _spmm_body,
        out_type=jax.ShapeDtypeStruct((NC, NPAD, D), jnp.float32),
        mesh=_mesh(),
        scratch_types=[
            pltpu.VMEM((NIB, K), jnp.int32),
            pltpu.VMEM((NIB, K), jnp.int32),
            pltpu.VMEM((NBUF, K, D), jnp.float32),
            pltpu.VMEM_SHARED((NPAD, D), jnp.float32),
            pltpu.SemaphoreType.DMA((NIB,)),
            pltpu.SemaphoreType.DMA((NBUF,)),
        ],
    )(feat_src, src3, dst3, zrows)

    out = pl.pallas_call(
        _out_body,
        out_shape=jax.ShapeDtypeStruct((N, D), jnp.float32),
    )(aggp, weight, degp[1][..., None], bias.reshape(1, D))
    return out


# speedup vs baseline: 11.4721x; 1.0829x over previous
import jax
import jax.numpy as jnp
from jax import lax
from jax.experimental import pallas as pl
from jax.experimental.pallas import tpu as pltpu
from jax.experimental.pallas import tpu_sc as plsc

N = 10000
NPAD = 10240
E = 320000
D = 128
NC = 2
NS = 16
NW = NC * NS
EW = E // NW
K = 80
NCH = EW // K
SL = NPAD // NS
NBUF = 4
NIB = 2 * NBUF
NMAIN = (NCH // NIB) * NIB
AG = 5
RB = 10


def _mesh():
    return plsc.VectorSubcoreMesh(
        core_axis_name="c", subcore_axis_name="s", num_cores=NC, num_subcores=NS
    )


def _fill_ones(ones):
    for i in range(K // 16):
        ones[pl.ds(i * 16, 16)] = jnp.ones((16,), jnp.float32)


def _deg_body(src3, dst3, zeros1, degp, sidx, didx, ones, sh_out, sh_in, sem):
    c = lax.axis_index("c")
    s = lax.axis_index("s")
    wid = s * NC + c
    pltpu.sync_copy(zeros1, sh_out.at[pl.ds(s * SL, SL)])
    pltpu.sync_copy(zeros1, sh_in.at[pl.ds(s * SL, SL)])
    _fill_ones(ones)
    pltpu.sync_copy(src3.at[wid], sidx)
    pltpu.sync_copy(dst3.at[wid], didx)
    plsc.subcore_barrier()

    def group(g, carry):
        for t in range(AG):
            j = g * AG + t
            pltpu.async_copy(ones, sh_out.at[sidx.at[j]], sem, add=True)
            pltpu.async_copy(ones, sh_in.at[didx.at[j]], sem, add=True)
        for t in range(AG):
            j = g * AG + t
            pltpu.make_async_copy(ones, sh_out.at[sidx.at[j]], sem).wait()
            pltpu.make_async_copy(ones, sh_in.at[didx.at[j]], sem).wait()
        return carry

    lax.fori_loop(0, NCH // AG, group, 0)
    plsc.subcore_barrier()
    pltpu.sync_copy(sh_out.at[pl.ds(s * SL, SL)], degp.at[0, c, pl.ds(s * SL, SL)])
    pltpu.sync_copy(sh_in.at[pl.ds(s * SL, SL)], degp.at[1, c, pl.ds(s * SL, SL)])


def _spmm_body(table, src3, dst3, aggp, sbuf, dbuf, rows, sh_acc,
               isem, gsem):
    c = lax.axis_index("c")
    s = lax.axis_index("s")
    wid = s * NC + c
    zv = jnp.zeros((16,), jnp.float32)

    def zrow(i, carry):
        for kk in range(D // 16):
            rows[0, i, pl.ds(kk * 16, 16)] = zv
        return carry

    lax.fori_loop(0, K, zrow, 0)
    for r in range(SL // K):
        pltpu.sync_copy(rows.at[0], sh_acc.at[pl.ds(s * SL + r * K, K)])
    plsc.subcore_barrier()

    def fire_idx(j, slot):
        pltpu.async_copy(src3.at[wid, j], sbuf.at[slot], isem.at[slot])
        pltpu.async_copy(dst3.at[wid, j], dbuf.at[slot], isem.at[slot])

    def wait_idx(slot):
        pltpu.make_async_copy(src3.at[wid, 0], sbuf.at[slot], isem.at[slot]).wait()
        pltpu.make_async_copy(src3.at[wid, 0], dbuf.at[slot], isem.at[slot]).wait()

    def fire_gather(slot, b):
        pltpu.async_copy(table.at[sbuf.at[slot]], rows.at[b], gsem.at[b])

    def wait_gather(b):
        pltpu.make_async_copy(table.at[pl.ds(0, K)], rows.at[b], gsem.at[b]).wait()

    for b in range(NBUF):
        fire_idx(b, b)
    for b in range(NBUF):
        wait_idx(b)
        fire_gather(b, b)
        fire_idx(b + NBUF, b + NBUF)

    def outer(g, carry):
        for ib in range(NIB):
            b = ib % NBUF
            j = g * NIB + ib
            wait_gather(b)
            pltpu.sync_copy(rows.at[b], sh_acc.at[dbuf.at[ib]], add=True)

            @pl.when(j + NIB < NCH)
            def _refill():
                fire_idx(j + NIB, ib)

            nslot = (ib + NBUF) % NIB
            wait_idx(nslot)
            fire_gather(nslot, b)
        return carry

    lax.fori_loop(0, NMAIN // NIB, outer, 0)

    for t in range(NCH - NMAIN):
        j = NMAIN + t
        ib = j % NIB
        b = ib % NBUF
        wait_gather(b)
        pltpu.sync_copy(rows.at[b], sh_acc.at[dbuf.at[ib]], add=True)
        nxt = j + NBUF
        if nxt < NCH:
            nslot = nxt % NIB
            wait_idx(nslot)
            fire_gather(nslot, b)

    plsc.subcore_barrier()
    pltpu.sync_copy(sh_acc.at[pl.ds(s * SL, SL)], aggp.at[c, pl.ds(s * SL, SL)])


def _src_norm_body(f_ref, d_ref, o_ref):
    deg = d_ref[0, :, :] + d_ref[1, :, :]
    norm = lax.rsqrt(jnp.maximum(deg, 1.0))
    o_ref[...] = f_ref[...] * norm


def _out_body(a_ref, w_ref, d_ref, b_ref, o_ref):
    agg = a_ref[0, :, :] + a_ref[1, :, :]
    rst = jnp.dot(agg, w_ref[...], preferred_element_type=jnp.float32)
    deg = d_ref[0, :, :] + d_ref[1, :, :]
    norm = lax.rsqrt(jnp.maximum(deg, 1.0))
    o_ref[...] = rst * norm + b_ref[...]


@jax.jit
def kernel(feat, edge_index, weight, bias):
    src3 = edge_index[0].astype(jnp.int32).reshape(NW, NCH, K)
    dst3 = edge_index[1].astype(jnp.int32).reshape(NW, NCH, K)
    zeros1 = jnp.zeros((SL,), jnp.float32)

    degp = pl.kernel(
        _deg_body,
        out_type=jax.ShapeDtypeStruct((2, NC, NPAD), jnp.float32),
        mesh=_mesh(),
        scratch_types=[
            pltpu.VMEM((NCH, K), jnp.int32),
            pltpu.VMEM((NCH, K), jnp.int32),
            pltpu.VMEM((K,), jnp.float32),
            pltpu.VMEM_SHARED((NPAD,), jnp.float32),
            pltpu.VMEM_SHARED((NPAD,), jnp.float32),
            pltpu.SemaphoreType.DMA,
        ],
    )(src3, dst3, zeros1)

    nb = N // RB
    feat_src = pl.pallas_call(
        _src_norm_body,
        grid=(RB,),
        in_specs=[
            pl.BlockSpec((nb, D), lambda i: (i, 0)),
            pl.BlockSpec((NC, nb, 1), lambda i: (0, i, 0)),
        ],
        out_specs=pl.BlockSpec((nb, D), lambda i: (i, 0)),
        out_shape=jax.ShapeDtypeStruct((N, D), jnp.float32),
    )(feat, degp[0][:, :, None])

    aggp = pl.kernel(
        _spmm_body,
        out_type=jax.ShapeDtypeStruct((NC, NPAD, D), jnp.float32),
        mesh=_mesh(),
        scratch_types=[
            pltpu.VMEM((NIB, K), jnp.int32),
            pltpu.VMEM((NIB, K), jnp.int32),
            pltpu.VMEM((NBUF, K, D), jnp.float32),
            pltpu.VMEM_SHARED((NPAD, D), jnp.float32),
            pltpu.SemaphoreType.DMA((NIB,)),
            pltpu.SemaphoreType.DMA((NBUF,)),
        ],
    )(feat_src, src3, dst3)

    out = pl.pallas_call(
        _out_body,
        grid=(RB,),
        in_specs=[
            pl.BlockSpec((NC, nb, D), lambda i: (0, i, 0)),
            pl.BlockSpec((D, D), lambda i: (0, 0)),
            pl.BlockSpec((NC, nb, 1), lambda i: (0, i, 0)),
            pl.BlockSpec((1, D), lambda i: (0, 0)),
        ],
        out_specs=pl.BlockSpec((nb, D), lambda i: (i, 0)),
        out_shape=jax.ShapeDtypeStruct((N, D), jnp.float32),
    )(aggp, weight, degp[1][:, :, None], bias.reshape(1, D))
    return out
